# baseline (speedup 1.0000x reference)
import functools

import jax
import jax.numpy as jnp
import numpy as np
from jax import lax
from jax.experimental import pallas as pl
from jax.experimental.pallas import tpu as pltpu
from jax.experimental.pallas import tpu_sc as plsc

_L_MAX = 3
_N_MAX = 32
_R_CUT = 5.0

_NC = 2
_NS = 16
_NW = _NC * _NS


def _sc_gather_body(per_w, ch, nch,
                    pos_hbm, ctr_hbm, nbr_hbm, out_hbm,
                    idx_c, idx_n, rows_c, rows_n, sem_c, sem_n):
    wid = lax.axis_index("s") * _NC + lax.axis_index("c")
    base = wid * per_w

    def chunk(i, carry):
        off = base + i * ch
        pltpu.sync_copy(ctr_hbm.at[pl.ds(off, ch)], idx_c)
        pltpu.sync_copy(nbr_hbm.at[pl.ds(off, ch)], idx_n)
        cn = pltpu.async_copy(pos_hbm.at[idx_n], rows_n, sem_n)
        cc = pltpu.async_copy(pos_hbm.at[idx_c], rows_c, sem_c)
        cn.wait()
        cc.wait()

        def sub(j, c2):
            rows_n[j] = rows_n[j] - rows_c[j]
            return c2

        lax.fori_loop(0, ch, sub, 0, unroll=4)
        pltpu.sync_copy(rows_n, out_hbm.at[pl.ds(off, ch)])
        return carry

    lax.fori_loop(0, nch, chunk, 0)


def _sc_gather(pos16, ctr, nbr):
    e = ctr.shape[0]
    per_w = e // _NW
    ch = 2000
    while per_w % ch:
        ch //= 2
    nch = per_w // ch
    mesh = plsc.VectorSubcoreMesh(core_axis_name="c", subcore_axis_name="s")
    return pl.kernel(
        functools.partial(_sc_gather_body, per_w, ch, nch),
        out_type=jax.ShapeDtypeStruct((e, 16), jnp.float32),
        mesh=mesh,
        scratch_types=[
            pltpu.VMEM((ch,), jnp.int32),
            pltpu.VMEM((ch,), jnp.int32),
            pltpu.VMEM((ch, 16), jnp.float32),
            pltpu.VMEM((ch, 16), jnp.float32),
            pltpu.SemaphoreType.DMA,
            pltpu.SemaphoreType.DMA,
        ],
        compiler_params=pltpu.CompilerParams(use_tc_tiling_on_sc=False),
    )(pos16, ctr, nbr)


_C0 = float(0.5 * np.sqrt(1.0 / np.pi))
_C1 = float(np.sqrt(3.0 / (4.0 * np.pi)))
_C2A = float(0.5 * np.sqrt(15.0 / np.pi))
_C2B = float(0.25 * np.sqrt(5.0 / np.pi))
_C2C = float(0.25 * np.sqrt(15.0 / np.pi))
_C3A = float(0.25 * np.sqrt(35.0 / (2.0 * np.pi)))
_C3B = float(0.5 * np.sqrt(105.0 / np.pi))
_C3C = float(0.25 * np.sqrt(21.0 / (2.0 * np.pi)))
_C3D = float(0.25 * np.sqrt(7.0 / np.pi))
_C3E = float(0.25 * np.sqrt(105.0 / np.pi))


def _tc_body(v_ref, o0, o1, o2, o3):
    v = v_ref[...]
    x = v[:, 0:1]
    y = v[:, 1:2]
    z = v[:, 2:3]
    r2 = x * x + y * y + z * z + 1e-12
    r = jnp.sqrt(r2)
    inv_r = 1.0 / r

    ni = lax.broadcasted_iota(jnp.int32, (1, _N_MAX), 1) + 1
    npi = ni.astype(jnp.float32) * np.float32(np.pi / _R_CUT)
    base = np.sqrt(2.0 / _R_CUT) * jnp.sin(r * npi) / (r + 1e-12)
    t = jnp.clip(r * (1.0 / _R_CUT), 0.0, 1.0)
    cut = 0.5 * (jnp.cos(np.pi * t) + 1.0)
    b0 = base * cut
    s = jnp.clip(r * (1.0 / _R_CUT), 1e-12, 1.0)
    b1 = b0 * s
    b2 = b1 * s
    b3 = b2 * s

    ux = x * inv_r
    uy = y * inv_r
    uz = z * inv_r
    xx = ux * ux
    yy = uy * uy
    zz = uz * uz

    o0[...] = b0 * _C0

    o1[:, 0:32] = b1 * (_C1 * uy)
    o1[:, 32:64] = b1 * (_C1 * uz)
    o1[:, 64:96] = b1 * (_C1 * ux)

    o2[:, 0:32] = b2 * (_C2A * ux * uy)
    o2[:, 32:64] = b2 * (_C2A * uy * uz)
    o2[:, 64:96] = b2 * (_C2B * (3.0 * zz - 1.0))
    o2[:, 96:128] = b2 * (_C2A * ux * uz)
    o2[:, 128:160] = b2 * (_C2C * (xx - yy))

    o3[:, 0:32] = b3 * (_C3A * uy * (3.0 * xx - yy))
    o3[:, 32:64] = b3 * (_C3B * ux * uy * uz)
    o3[:, 64:96] = b3 * (_C3C * uy * (5.0 * zz - 1.0))
    o3[:, 96:128] = b3 * (_C3D * uz * (5.0 * zz - 3.0))
    o3[:, 128:160] = b3 * (_C3C * ux * (5.0 * zz - 1.0))
    o3[:, 160:192] = b3 * (_C3E * uz * (xx - yy))
    o3[:, 192:224] = b3 * (_C3A * ux * (xx - yy))


def _tc_dense(vec16, interpret=False):
    e = vec16.shape[0]
    be = 1024
    while e % be:
        be //= 2
    widths = [(2 * l + 1) * _N_MAX for l in range(_L_MAX + 1)]
    return pl.pallas_call(
        _tc_body,
        grid=(e // be,),
        in_specs=[pl.BlockSpec((be, 16), lambda i: (i, 0))],
        out_specs=tuple(
            pl.BlockSpec((be, w), lambda i: (i, 0)) for w in widths),
        out_shape=tuple(
            jax.ShapeDtypeStruct((e, w), jnp.float32) for w in widths),
        compiler_params=pltpu.CompilerParams(
            dimension_semantics=("arbitrary",)),
        interpret=interpret,
    )(vec16)


def kernel(positions, edge_index):
    n = positions.shape[0]
    e = edge_index.shape[1]
    pos16 = jnp.zeros((n, 16), positions.dtype).at[:, :3].set(positions)
    ei = edge_index.astype(jnp.int32)
    vec16 = _sc_gather(pos16, ei[0], ei[1])
    o0, o1, o2, o3 = _tc_dense(vec16)
    return (o0.reshape(e, 1, 32), o1.reshape(e, 3, 32),
            o2.reshape(e, 5, 32), o3.reshape(e, 7, 32))


# speedup vs baseline: 2.5957x; 2.5957x over previous
import functools

import jax
import jax.numpy as jnp
import numpy as np
from jax import lax
from jax.experimental import pallas as pl
from jax.experimental.pallas import tpu as pltpu
from jax.experimental.pallas import tpu_sc as plsc

_L_MAX = 3
_N_MAX = 32
_R_CUT = 5.0

_NC = 2
_NS = 16
_NW = _NC * _NS


def _sc_gather_body(per_w, ch, nch,
                    pos_hbm, ctr_hbm, nbr_hbm, out_hbm,
                    idx_c, idx_n, rows_c, rows_n, sem_c, sem_n):
    wid = lax.axis_index("s") * _NC + lax.axis_index("c")
    base = wid * per_w

    def chunk(i, carry):
        off = base + i * ch
        pltpu.sync_copy(ctr_hbm.at[pl.ds(off, ch)], idx_c)
        pltpu.sync_copy(nbr_hbm.at[pl.ds(off, ch)], idx_n)
        cn = pltpu.async_copy(pos_hbm.at[idx_n], rows_n, sem_n)
        cc = pltpu.async_copy(pos_hbm.at[idx_c], rows_c, sem_c)
        cn.wait()
        cc.wait()

        def sub(j, c2):
            rows_n[j] = rows_n[j] - rows_c[j]
            return c2

        lax.fori_loop(0, ch, sub, 0, unroll=4)
        pltpu.sync_copy(rows_n, out_hbm.at[pl.ds(off, ch)])
        return carry

    lax.fori_loop(0, nch, chunk, 0)


def _sc_gather(pos16, ctr, nbr):
    e = ctr.shape[0]
    per_w = e // _NW
    ch = 2000
    while per_w % ch:
        ch //= 2
    nch = per_w // ch
    mesh = plsc.VectorSubcoreMesh(core_axis_name="c", subcore_axis_name="s")
    return pl.kernel(
        functools.partial(_sc_gather_body, per_w, ch, nch),
        out_type=jax.ShapeDtypeStruct((e, 16), jnp.float32),
        mesh=mesh,
        scratch_types=[
            pltpu.VMEM((ch,), jnp.int32),
            pltpu.VMEM((ch,), jnp.int32),
            pltpu.VMEM((ch, 16), jnp.float32),
            pltpu.VMEM((ch, 16), jnp.float32),
            pltpu.SemaphoreType.DMA,
            pltpu.SemaphoreType.DMA,
        ],
        compiler_params=pltpu.CompilerParams(use_tc_tiling_on_sc=False),
    )(pos16, ctr, nbr)


_C0 = float(0.5 * np.sqrt(1.0 / np.pi))
_C1 = float(np.sqrt(3.0 / (4.0 * np.pi)))
_C2A = float(0.5 * np.sqrt(15.0 / np.pi))
_C2B = float(0.25 * np.sqrt(5.0 / np.pi))
_C2C = float(0.25 * np.sqrt(15.0 / np.pi))
_C3A = float(0.25 * np.sqrt(35.0 / (2.0 * np.pi)))
_C3B = float(0.5 * np.sqrt(105.0 / np.pi))
_C3C = float(0.25 * np.sqrt(21.0 / (2.0 * np.pi)))
_C3D = float(0.25 * np.sqrt(7.0 / np.pi))
_C3E = float(0.25 * np.sqrt(105.0 / np.pi))


def _tc_body(v_ref, sel_ref, tile_ref, o0, o1, o2, o3):
    vt = jnp.transpose(v_ref[...])
    x = vt[0:1, :]
    y = vt[1:2, :]
    z = vt[2:3, :]
    r2 = x * x + y * y + z * z + 1e-12
    r = jnp.sqrt(r2)
    inv_r = 1.0 / r
    invden = np.float32(np.sqrt(2.0 / _R_CUT)) / (r + 1e-12)

    t = jnp.minimum(r * np.float32(1.0 / _R_CUT), 1.0)
    cut = 0.5 * jnp.cos(np.float32(np.pi) * t) + 0.5
    s = jnp.clip(r * np.float32(1.0 / _R_CUT), 1e-12, 1.0)
    cs0 = cut * invden
    cs1 = cs0 * s
    cs2 = cs1 * s
    cs3 = cs2 * s

    ux = x * inv_r
    uy = y * inv_r
    uz = z * inv_r
    xx = ux * ux
    yy = uy * uy
    zz = uz * uz

    w = [
        cs0 * _C0,
        cs1 * (_C1 * uy),
        cs1 * (_C1 * uz),
        cs1 * (_C1 * ux),
        cs2 * (_C2A * ux * uy),
        cs2 * (_C2A * uy * uz),
        cs2 * (_C2B * (3.0 * zz - 1.0)),
        cs2 * (_C2A * ux * uz),
        cs2 * (_C2C * (xx - yy)),
        cs3 * (_C3A * uy * (3.0 * xx - yy)),
        cs3 * (_C3B * ux * uy * uz),
        cs3 * (_C3C * uy * (5.0 * zz - 1.0)),
        cs3 * (_C3D * uz * (5.0 * zz - 3.0)),
        cs3 * (_C3C * ux * (5.0 * zz - 1.0)),
        cs3 * (_C3E * uz * (xx - yy)),
        cs3 * (_C3A * ux * (xx - yy)),
    ]
    wrows = jnp.concatenate(w, axis=0)

    theta = r * np.float32(np.pi / _R_CUT)
    ncol = (lax.broadcasted_iota(jnp.int32, (_N_MAX, 1), 0) + 1).astype(
        jnp.float32)
    sint = jnp.sin(ncol * theta)

    dn = (((0,), (0,)), ((), ()))
    wbig = lax.dot_general(wrows, sel_ref[...], dn,
                           preferred_element_type=jnp.float32)
    sbig = lax.dot_general(sint, tile_ref[...], dn,
                           preferred_element_type=jnp.float32)
    out = wbig * sbig

    o0[...] = out[:, 0:32]
    o1[...] = out[:, 32:128]
    o2[...] = out[:, 128:288]
    o3[...] = out[:, 288:512]


def _sel_consts():
    nw = 16 * _N_MAX
    sel = np.zeros((16, nw), np.float32)
    for m in range(16):
        sel[m, m * _N_MAX:(m + 1) * _N_MAX] = 1.0
    tile = np.zeros((_N_MAX, nw), np.float32)
    for m in range(16):
        tile[np.arange(_N_MAX), m * _N_MAX + np.arange(_N_MAX)] = 1.0
    return sel, tile


def _tc_dense(vec16, interpret=False):
    e = vec16.shape[0]
    be = 1024
    while e % be:
        be //= 2
    widths = [(2 * l + 1) * _N_MAX for l in range(_L_MAX + 1)]
    sel, tile = _sel_consts()
    return pl.pallas_call(
        _tc_body,
        grid=(e // be,),
        in_specs=[
            pl.BlockSpec((be, 16), lambda i: (i, 0)),
            pl.BlockSpec((16, 16 * _N_MAX), lambda i: (0, 0)),
            pl.BlockSpec((_N_MAX, 16 * _N_MAX), lambda i: (0, 0)),
        ],
        out_specs=tuple(
            pl.BlockSpec((be, w), lambda i: (i, 0)) for w in widths),
        out_shape=tuple(
            jax.ShapeDtypeStruct((e, w), jnp.float32) for w in widths),
        compiler_params=pltpu.CompilerParams(
            dimension_semantics=("arbitrary",)),
        interpret=interpret,
    )(vec16, jnp.asarray(sel), jnp.asarray(tile))


def kernel(positions, edge_index):
    n = positions.shape[0]
    e = edge_index.shape[1]
    pos16 = jnp.zeros((n, 16), positions.dtype).at[:, :3].set(positions)
    ei = edge_index.astype(jnp.int32)
    vec16 = _sc_gather(pos16, ei[0], ei[1])
    o0, o1, o2, o3 = _tc_dense(vec16)
    return (o0.reshape(e, 1, 32), o1.reshape(e, 3, 32),
            o2.reshape(e, 5, 32), o3.reshape(e, 7, 32))


# speedup vs baseline: 9.9316x; 3.8262x over previous
import functools

import jax
import jax.numpy as jnp
import numpy as np
from jax import lax
from jax.experimental import pallas as pl
from jax.experimental.pallas import tpu as pltpu
from jax.experimental.pallas import tpu_sc as plsc

_L_MAX = 3
_N_MAX = 32
_R_CUT = 5.0

_NC = 2
_NS = 16
_NW = _NC * _NS


def _sc_gather_body(n, per_w, ch, nch,
                    pos_hbm, ctr_hbm, nbr_hbm, out_hbm,
                    px, py, pz, idx_c, idx_n, vx, vy, vz, sem):
    wid = lax.axis_index("s") * _NC + lax.axis_index("c")
    base = wid * per_w
    cx = pltpu.async_copy(pos_hbm.at[0], px, sem)
    cy = pltpu.async_copy(pos_hbm.at[1], py, sem)
    cz = pltpu.async_copy(pos_hbm.at[2], pz, sem)
    cx.wait()
    cy.wait()
    cz.wait()

    def chunk(i, carry):
        off = base + i * ch
        pltpu.sync_copy(ctr_hbm.at[pl.ds(off, ch)], idx_c)
        pltpu.sync_copy(nbr_hbm.at[pl.ds(off, ch)], idx_n)

        def sub(j, c2_):
            sl = pl.ds(j * 16, 16)
            ic = idx_c[sl]
            inb = idx_n[sl]
            vx[sl] = plsc.load_gather(px, [inb]) - plsc.load_gather(px, [ic])
            vy[sl] = plsc.load_gather(py, [inb]) - plsc.load_gather(py, [ic])
            vz[sl] = plsc.load_gather(pz, [inb]) - plsc.load_gather(pz, [ic])
            return c2_

        lax.fori_loop(0, ch // 16, sub, 0, unroll=4)
        pltpu.sync_copy(vx, out_hbm.at[0, pl.ds(off, ch)])
        pltpu.sync_copy(vy, out_hbm.at[1, pl.ds(off, ch)])
        pltpu.sync_copy(vz, out_hbm.at[2, pl.ds(off, ch)])
        return carry

    lax.fori_loop(0, nch, chunk, 0)


def _sc_gather(pos3, ctr, nbr):
    n = pos3.shape[1]
    e = ctr.shape[0]
    per_w = e // _NW
    ch = 2000
    while per_w % ch or ch % 16:
        ch //= 2
    nch = per_w // ch
    mesh = plsc.VectorSubcoreMesh(core_axis_name="c", subcore_axis_name="s")
    return pl.kernel(
        functools.partial(_sc_gather_body, n, per_w, ch, nch),
        out_type=jax.ShapeDtypeStruct((3, e), jnp.float32),
        mesh=mesh,
        scratch_types=[
            pltpu.VMEM((n,), jnp.float32),
            pltpu.VMEM((n,), jnp.float32),
            pltpu.VMEM((n,), jnp.float32),
            pltpu.VMEM((ch,), jnp.int32),
            pltpu.VMEM((ch,), jnp.int32),
            pltpu.VMEM((ch,), jnp.float32),
            pltpu.VMEM((ch,), jnp.float32),
            pltpu.VMEM((ch,), jnp.float32),
            pltpu.SemaphoreType.DMA,
        ],
        compiler_params=pltpu.CompilerParams(use_tc_tiling_on_sc=False,
                                             needs_layout_passes=False),
    )(pos3, ctr, nbr)


_C0 = float(0.5 * np.sqrt(1.0 / np.pi))
_C1 = float(np.sqrt(3.0 / (4.0 * np.pi)))
_C2A = float(0.5 * np.sqrt(15.0 / np.pi))
_C2B = float(0.25 * np.sqrt(5.0 / np.pi))
_C2C = float(0.25 * np.sqrt(15.0 / np.pi))
_C3A = float(0.25 * np.sqrt(35.0 / (2.0 * np.pi)))
_C3B = float(0.5 * np.sqrt(105.0 / np.pi))
_C3C = float(0.25 * np.sqrt(21.0 / (2.0 * np.pi)))
_C3D = float(0.25 * np.sqrt(7.0 / np.pi))
_C3E = float(0.25 * np.sqrt(105.0 / np.pi))


def _tc_body(v_ref, o0, o1, o2, o3):
    vt = v_ref[...]
    x = vt[0:1, :]
    y = vt[1:2, :]
    z = vt[2:3, :]
    r2 = x * x + y * y + z * z + 1e-12
    r = jnp.sqrt(r2)
    inv_r = 1.0 / r
    invden = np.float32(np.sqrt(2.0 / _R_CUT)) / (r + 1e-12)

    t = jnp.minimum(r * np.float32(1.0 / _R_CUT), 1.0)
    cut = 0.5 * jnp.cos(np.float32(np.pi) * t) + 0.5
    s = jnp.clip(r * np.float32(1.0 / _R_CUT), 1e-12, 1.0)
    cs0 = cut * invden
    cs1 = cs0 * s
    cs2 = cs1 * s
    cs3 = cs2 * s

    ux = x * inv_r
    uy = y * inv_r
    uz = z * inv_r
    xx = ux * ux
    yy = uy * uy
    zz = uz * uz

    w = [
        cs0 * _C0,
        cs1 * (_C1 * uy),
        cs1 * (_C1 * uz),
        cs1 * (_C1 * ux),
        cs2 * (_C2A * ux * uy),
        cs2 * (_C2A * uy * uz),
        cs2 * (_C2B * (3.0 * zz - 1.0)),
        cs2 * (_C2A * ux * uz),
        cs2 * (_C2C * (xx - yy)),
        cs3 * (_C3A * uy * (3.0 * xx - yy)),
        cs3 * (_C3B * ux * uy * uz),
        cs3 * (_C3C * uy * (5.0 * zz - 1.0)),
        cs3 * (_C3D * uz * (5.0 * zz - 3.0)),
        cs3 * (_C3C * ux * (5.0 * zz - 1.0)),
        cs3 * (_C3E * uz * (xx - yy)),
        cs3 * (_C3A * ux * (xx - yy)),
    ]

    theta = r * np.float32(np.pi / _R_CUT)
    ncol = (lax.broadcasted_iota(jnp.int32, (_N_MAX, 1), 0) + 1).astype(
        jnp.float32)
    sint = jnp.sin(ncol * theta)

    outs = [o0, o1, o2, o3]
    m = 0
    for l in range(_L_MAX + 1):
        for mm in range(2 * l + 1):
            outs[l][mm * _N_MAX:(mm + 1) * _N_MAX, :] = sint * w[m]
            m += 1


def _tc_dense(vec3, interpret=False):
    e = vec3.shape[1]
    be = 1024
    while e % be:
        be //= 2
    widths = [(2 * l + 1) * _N_MAX for l in range(_L_MAX + 1)]
    return pl.pallas_call(
        _tc_body,
        grid=(e // be,),
        in_specs=[pl.BlockSpec((3, be), lambda i: (0, i))],
        out_specs=tuple(
            pl.BlockSpec((w, be), lambda i: (0, i)) for w in widths),
        out_shape=tuple(
            jax.ShapeDtypeStruct((w, e), jnp.float32) for w in widths),
        compiler_params=pltpu.CompilerParams(
            dimension_semantics=("arbitrary",)),
        interpret=interpret,
    )(vec3)


def kernel(positions, edge_index):
    n = positions.shape[0]
    e = edge_index.shape[1]
    pos3 = jnp.transpose(positions)
    ei = edge_index.astype(jnp.int32)
    vec3 = _sc_gather(pos3, ei[0], ei[1])
    outs = _tc_dense(vec3)
    return tuple(
        o.reshape(2 * l + 1, _N_MAX, e).transpose(2, 0, 1)
        for l, o in enumerate(outs))
